# baseline (speedup 1.0000x reference)
import math
from functools import partial

import jax
import jax.numpy as jnp
from jax import lax
from jax.experimental import pallas as pl
from jax.experimental.pallas import tpu as pltpu

_EPS = 1e-5


def _ru(x, m):
    return (x + m - 1) // m * m


def _pick_tk(k):
    kp_best, tk_best = None, None
    for tk in (512, 384, 256, 128):
        kp = _ru(k, tk)
        if kp_best is None or kp < kp_best:
            kp_best, tk_best = kp, tk
    return kp_best, tk_best


def _mm_body(tk, aff, in_relu, bn, has_bias, op_dtype, *refs):
    refs = list(refs)
    a_ref = refs.pop(0)
    b_ref = refs.pop(0)
    s_ref = refs.pop(0) if aff else None
    h_ref = refs.pop(0) if aff else None
    c_ref = refs.pop(0) if has_bias else None
    o_ref = refs.pop(0)
    st_ref = refs.pop(0) if bn else None
    acc_ref = refs.pop(0)

    k = pl.program_id(2)
    nk = pl.num_programs(2)

    @pl.when(k == 0)
    def _():
        acc_ref[...] = jnp.zeros_like(acc_ref)

    a = a_ref[...]
    if aff:
        v = a.astype(jnp.float32) * s_ref[...] + h_ref[...]
        if in_relu:
            v = jnp.maximum(v, 0.0)
        a = v.astype(op_dtype)
    off = pl.multiple_of(k * tk, tk)
    acc_ref[...] += jnp.dot(a, b_ref[pl.ds(off, tk), :],
                            preferred_element_type=jnp.float32)

    @pl.when(k == nk - 1)
    def _():
        y = acc_ref[...]
        if bn:
            sm = jnp.sum(y, axis=0, keepdims=True)
            sq = jnp.sum(y * y, axis=0, keepdims=True)
            st_ref[...] = jnp.concatenate(
                [sm, sq, jnp.zeros((6, y.shape[1]), jnp.float32)], axis=0)
        if has_bias:
            y = y + c_ref[...]
        o_ref[...] = y.astype(o_ref.dtype)


def _mm(a, b, *, in_scale=None, in_shift=None, in_relu=True, bn=True,
        bias=None, op_dtype=jnp.bfloat16, out_dtype=jnp.float32):
    M, K = a.shape
    N = b.shape[1]
    aff = in_scale is not None

    Kp, tk = _pick_tk(K)
    Mp = _ru(M, 8)
    tm = Mp if Mp <= 512 else 512
    Mp = _ru(Mp, tm)
    Np = _ru(N, 128)
    tn = 256 if Np % 256 == 0 else 128

    if not aff:
        a = a.astype(op_dtype)
    if (Mp, Kp) != (M, K):
        a = jnp.pad(a, ((0, Mp - M), (0, Kp - K)))
    b = b.astype(op_dtype)
    if (Kp, Np) != b.shape:
        b = jnp.pad(b, ((0, Kp - b.shape[0]), (0, Np - N)))

    ni, nj, nk = Mp // tm, Np // tn, Kp // tk

    args = [a, b]
    in_specs = [pl.BlockSpec((tm, tk), lambda i, j, k: (i, k)),
                pl.BlockSpec((Kp, tn), lambda i, j, k: (0, j))]
    if aff:
        s = jnp.pad(in_scale.reshape(1, K), ((0, 0), (0, Kp - K)))
        h = jnp.pad(in_shift.reshape(1, K), ((0, 0), (0, Kp - K)))
        args += [s, h]
        in_specs += [pl.BlockSpec((1, tk), lambda i, j, k: (0, k)),
                     pl.BlockSpec((1, tk), lambda i, j, k: (0, k))]
    if bias is not None:
        args.append(jnp.pad(bias.astype(jnp.float32).reshape(1, N),
                            ((0, 0), (0, Np - N))))
        in_specs.append(pl.BlockSpec((1, tn), lambda i, j, k: (0, j)))

    out_shape = [jax.ShapeDtypeStruct((Mp, Np), out_dtype)]
    out_specs = [pl.BlockSpec((tm, tn), lambda i, j, k: (i, j))]
    if bn:
        out_shape.append(jax.ShapeDtypeStruct((ni * 8, Np), jnp.float32))
        out_specs.append(pl.BlockSpec((8, tn), lambda i, j, k: (i, j)))

    res = pl.pallas_call(
        partial(_mm_body, tk, aff, in_relu, bn, bias is not None, op_dtype),
        grid=(ni, nj, nk),
        in_specs=in_specs,
        out_specs=out_specs,
        out_shape=out_shape,
        scratch_shapes=[pltpu.VMEM((tm, tn), jnp.float32)],
        compiler_params=pltpu.CompilerParams(
            dimension_semantics=("parallel", "parallel", "arbitrary"),
            vmem_limit_bytes=48 * 1024 * 1024),
    )(*args)
    if bn:
        y, st = res
        st = st.reshape(ni, 8, Np)[:, :2, :].sum(axis=0)
        return y[:M] if Mp != M else y, st
    return res[0][:M] if Mp != M else res[0]


def _dconv_body(chunks, tk, dil, tr, W, H, aff, *refs):
    refs = list(refs)
    x_ref = refs.pop(0)
    w_ref = refs.pop(0)
    s_ref = refs.pop(0) if aff else None
    h_ref = refs.pop(0) if aff else None
    scr_ref = refs.pop()
    st_ref = refs.pop()
    o_ref = refs.pop()

    i = pl.program_id(2)
    tn = o_ref.shape[3]
    masks = {}
    acc = jnp.zeros((tr * W, tn), jnp.float32)
    for k, pieces, ztail in chunks:
        ops = []
        for (ky, kx, c0, c1) in pieces:
            cs = c1 - c0
            xs = x_ref[0, pl.ds(i * tr + ky * dil, tr),
                       pl.ds(kx * dil, W), c0:c1]
            xs = xs.reshape(tr * W, cs)
            if aff:
                v = xs.astype(jnp.float32) * s_ref[0, c0:c1] + h_ref[0, c0:c1]
                xs = jnp.maximum(v, 0.0).astype(jnp.bfloat16)
                if (ky, kx) not in masks:
                    fi = lax.broadcasted_iota(jnp.int32, (tr * W, 1), 0)
                    gr = fi // W + (i * tr + ky * dil)
                    gc = fi % W + kx * dil
                    masks[(ky, kx)] = ((gr >= dil) & (gr < H + dil)
                                       & (gc >= dil) & (gc < W + dil))
                xs = jnp.where(masks[(ky, kx)], xs, jnp.bfloat16(0))
            ops.append(xs)
        if len(ops) == 1 and not ztail:
            blk = ops[0]
        else:
            ofs = 0
            for xs in ops:
                scr_ref[:, ofs:ofs + xs.shape[1]] = xs
                ofs += xs.shape[1]
            if ztail:
                scr_ref[:, tk - ztail:] = jnp.zeros(
                    (tr * W, ztail), jnp.bfloat16)
            blk = scr_ref[...]
        acc = acc + jnp.dot(blk, w_ref[k * tk:(k + 1) * tk, :],
                            preferred_element_type=jnp.float32)
    o_ref[...] = acc.reshape(1, tr, W, tn)
    sm = jnp.sum(acc, axis=0, keepdims=True)
    sq = jnp.sum(acc * acc, axis=0, keepdims=True)
    st_ref[...] = jnp.concatenate(
        [sm, sq, jnp.zeros((6, tn), jnp.float32)], axis=0)


def _dconv(x, w, dil, *, in_scale=None, in_shift=None):
    N, H, W, C = x.shape
    cout = w.shape[0]
    cin = w.shape[1]
    aff = in_scale is not None

    Np = _ru(cout, 128)
    tn = 128 if (C >= 1024 or Np == 128) else 256
    tr = min(H, max(1, 512 // W))
    while H % tr:
        tr -= 1
    nir = H // tr
    Hp, Wp = H + 2 * dil, W + 2 * dil

    K = 9 * cin
    Kp, tk = _pick_tk(K)
    chunks = []
    for k in range(Kp // tk):
        lo, hi = k * tk, min((k + 1) * tk, K)
        pieces = []
        for t in range(9):
            a, b = max(lo, t * cin), min(hi, (t + 1) * cin)
            if a < b:
                pieces.append((t // 3, t % 3, a - t * cin, b - t * cin))
        chunks.append((k, pieces, (k + 1) * tk - max(hi, lo)))

    if aff:
        xp = jnp.pad(x, ((0, 0), (dil, dil), (dil, dil), (0, 0)))
        s = in_scale.reshape(1, C)
        h = in_shift.reshape(1, C)
    else:
        xp = jnp.pad(x, ((0, 0), (dil, dil), (dil, dil), (0, 0)))

    wt = jnp.transpose(w, (2, 3, 1, 0))
    wm = wt.reshape(9 * cin, cout).astype(jnp.bfloat16)
    wm = jnp.pad(wm, ((0, Kp - K), (0, Np - cout)))

    args = [xp, wm]
    in_specs = [pl.BlockSpec((1, Hp, Wp, C), lambda n, j, i: (n, 0, 0, 0)),
                pl.BlockSpec((Kp, tn), lambda n, j, i: (0, j))]
    if aff:
        args += [s, h]
        in_specs += [pl.BlockSpec((1, C), lambda n, j, i: (0, 0)),
                     pl.BlockSpec((1, C), lambda n, j, i: (0, 0))]

    y, st = pl.pallas_call(
        partial(_dconv_body, chunks, tk, dil, tr, W, H, aff),
        grid=(N, Np // tn, nir),
        in_specs=in_specs,
        out_specs=[pl.BlockSpec((1, tr, W, tn), lambda n, j, i: (n, i, 0, j)),
                   pl.BlockSpec((8, tn),
                                lambda n, j, i, _nir=nir: (n * _nir + i, j))],
        out_shape=[jax.ShapeDtypeStruct((N, H, W, Np), jnp.float32),
                   jax.ShapeDtypeStruct((N * nir * 8, Np), jnp.float32)],
        scratch_shapes=[pltpu.VMEM((tr * W, tk), jnp.bfloat16)],
        compiler_params=pltpu.CompilerParams(
            dimension_semantics=("parallel", "parallel", "arbitrary"),
            vmem_limit_bytes=60 * 1024 * 1024),
    )(*args)
    st = st.reshape(N * nir, 8, Np)[:, :2, :].sum(axis=0)
    return y, st


def _post_body(mode, *refs):
    refs = list(refs)
    y_ref, s_ref, h_ref = refs[:3]
    refs = refs[3:]
    v = y_ref[...] * s_ref[...] + h_ref[...]
    if mode == 1:
        v = v + refs[0][...].astype(jnp.float32)
    elif mode == 2:
        r = (refs[0][...] * refs[1][...] + refs[2][...]).astype(jnp.bfloat16)
        v = v + r.astype(jnp.float32)
    o_ref = refs[-1]
    o_ref[...] = jnp.maximum(v, 0.0).astype(o_ref.dtype)


def _post(y, s, h, *, ident=None, ds=None):
    M, Np = y.shape
    tm = 2048 if Np <= 512 else 512
    tm = min(tm, _ru(M, 8))
    Mp = _ru(M, tm)
    if Mp != M:
        y = jnp.pad(y, ((0, Mp - M), (0, 0)))

    mode = 0
    args = [y, s.reshape(1, Np), h.reshape(1, Np)]
    in_specs = [pl.BlockSpec((tm, Np), lambda i: (i, 0)),
                pl.BlockSpec((1, Np), lambda i: (0, 0)),
                pl.BlockSpec((1, Np), lambda i: (0, 0))]
    if ident is not None:
        mode = 1
        args.append(ident)
        in_specs.append(pl.BlockSpec((tm, Np), lambda i: (i, 0)))
    elif ds is not None:
        mode = 2
        yd, sd, hd = ds
        args += [yd, sd.reshape(1, Np), hd.reshape(1, Np)]
        in_specs += [pl.BlockSpec((tm, Np), lambda i: (i, 0)),
                     pl.BlockSpec((1, Np), lambda i: (0, 0)),
                     pl.BlockSpec((1, Np), lambda i: (0, 0))]

    out = pl.pallas_call(
        partial(_post_body, mode),
        grid=(Mp // tm,),
        in_specs=in_specs,
        out_specs=pl.BlockSpec((tm, Np), lambda i: (i, 0)),
        out_shape=jax.ShapeDtypeStruct((Mp, Np), jnp.bfloat16),
        compiler_params=pltpu.CompilerParams(
            dimension_semantics=("parallel",),
            vmem_limit_bytes=48 * 1024 * 1024),
    )(*args)
    return out[:M] if Mp != M else out


def _bn_affine(st, m):
    mean = st[0] / m
    var = jnp.maximum(st[1] / m - mean * mean, 0.0)
    s = lax.rsqrt(var + _EPS)
    return s, -mean * s


def _w1x1(w, kin):
    mat = jnp.transpose(w[:, :, 0, 0], (1, 0))
    return jnp.pad(mat, ((0, kin - mat.shape[0]), (0, 0)))


def _im2col(x, kh, kw, stride, padding):
    N, H, W, C = x.shape
    xp = jnp.pad(x, ((0, 0), (padding, padding), (padding, padding), (0, 0)))
    Ho = (H + 2 * padding - kh) // stride + 1
    Wo = (W + 2 * padding - kw) // stride + 1
    cols = [xp[:, i:i + stride * (Ho - 1) + 1:stride,
               j:j + stride * (Wo - 1) + 1:stride, :]
            for i in range(kh) for j in range(kw)]
    pat = jnp.stack(cols, axis=3)
    return pat.reshape(N * Ho * Wo, kh * kw * C), Ho, Wo


def _im2col_d(x, cin, dil):
    N, H, W, C = x.shape
    xp = jnp.pad(x[..., :cin], ((0, 0), (dil, dil), (dil, dil), (0, 0)))
    cols = [xp[:, i * dil:i * dil + H, j * dil:j * dil + W, :]
            for i in range(3) for j in range(3)]
    return jnp.stack(cols, axis=3).reshape(N * H * W, 9 * cin)


def _wk(w, kin):
    cout, cin, kh, kw = w.shape
    wt = jnp.transpose(w, (2, 3, 1, 0))
    wt = jnp.pad(wt, ((0, 0), (0, 0), (0, kin - cin), (0, 0)))
    return wt.reshape(kh * kw * kin, cout)


def _block_s1(x, w1, w2, w3, wd, dil):
    N, H, W, C = x.shape
    M = N * H * W
    xf = x.reshape(M, C)

    y1, st1 = _mm(xf, _w1x1(w1, C))
    s1, h1 = _bn_affine(st1, M)
    a1 = _post(y1, s1, h1).reshape(N, H, W, y1.shape[1])
    A2 = _im2col_d(a1, w2.shape[1], dil)
    y2, st2 = _mm(A2, _wk(w2, w2.shape[1]))
    s2, h2 = _bn_affine(st2, M)
    Np2 = y2.shape[1]
    a2 = _post(y2, s2, h2)
    y3, st3 = _mm(a2, _w1x1(w3, Np2))
    s3, h3 = _bn_affine(st3, M)

    if wd is not None:
        yd, std = _mm(xf, _w1x1(wd, C))
        sd, hd = _bn_affine(std, M)
        out = _post(y3, s3, h3, ds=(yd, sd, hd))
    else:
        out = _post(y3, s3, h3, ident=xf)
    return out.reshape(N, H, W, y3.shape[1])


def _block_s2(x, w1, w2, w3, wd):
    N, H, W, C = x.shape
    M = N * H * W
    xf = x.reshape(M, C)

    y1, st1 = _mm(xf, _w1x1(w1, C))
    s1, h1 = _bn_affine(st1, M)
    a1 = _post(y1, s1, h1).reshape(N, H, W, y1.shape[1])

    A2, Ho, Wo = _im2col(a1, 3, 3, 2, 1)
    Mo = N * Ho * Wo
    y2, st2 = _mm(A2, _wk(w2, a1.shape[3]))
    s2, h2 = _bn_affine(st2, Mo)
    y3, st3 = _mm(y2, _w1x1(w3, y2.shape[1]), in_scale=s2, in_shift=h2)
    s3, h3 = _bn_affine(st3, Mo)

    xs = x[:, ::2, ::2, :].reshape(Mo, C)
    yd, std = _mm(xs, _w1x1(wd, C))
    sd, hd = _bn_affine(std, Mo)
    out = _post(y3, s3, h3, ds=(yd, sd, hd))
    return out.reshape(N, Ho, Wo, y3.shape[1])


def _scan_blocks(x, w1s, w2s, w3s, dil):
    def body(hc, ws):
        return _block_s1(hc, ws[0], ws[1], ws[2], None, dil), None
    x, _ = lax.scan(body, x, (w1s, w2s, w3s))
    return x


def _interp_matrix(osz, isz):
    src = jnp.arange(osz, dtype=jnp.float32) * ((isz - 1) / (osz - 1))
    i0 = jnp.clip(jnp.floor(src).astype(jnp.int32), 0, isz - 1)
    i1 = jnp.clip(i0 + 1, 0, isz - 1)
    t = src - i0.astype(jnp.float32)
    oh0 = jax.nn.one_hot(i0, isz, dtype=jnp.float32)
    oh1 = jax.nn.one_hot(i1, isz, dtype=jnp.float32)
    return oh0 * (1.0 - t)[:, None] + oh1 * t[:, None]


def kernel(x, conv1, layer1_first_w1, layer1_first_w2, layer1_first_w3,
           layer1_first_wd, layer1_rest_w1, layer1_rest_w2, layer1_rest_w3,
           layer2_first_w1, layer2_first_w2, layer2_first_w3, layer2_first_wd,
           layer2_rest_w1, layer2_rest_w2, layer2_rest_w3,
           layer3_first_w1, layer3_first_w2, layer3_first_w3, layer3_first_wd,
           layer3_rest_w1, layer3_rest_w2, layer3_rest_w3,
           layer4_first_w1, layer4_first_w2, layer4_first_w3, layer4_first_wd,
           layer4_rest_w1, layer4_rest_w2, layer4_rest_w3,
           aspp_a1, aspp_a2, aspp_a3, aspp_a4, aspp_a5,
           aspp_c2, aspp_c3, aspp_c3b):
    N, _, H, W = x.shape
    xh = jnp.transpose(x, (0, 2, 3, 1)).astype(jnp.bfloat16)

    A0, Ho, Wo = _im2col(xh, 7, 7, 2, 3)
    M0 = N * Ho * Wo
    y0, st0 = _mm(A0, _wk(conv1, 3))
    s0, h0 = _bn_affine(st0, M0)
    a0 = _post(y0, s0, h0).reshape(N, Ho, Wo, y0.shape[1])
    xb = lax.reduce_window(a0, jnp.array(-jnp.inf, a0.dtype), lax.max,
                           (1, 3, 3, 1), (1, 2, 2, 1),
                           ((0, 0), (1, 1), (1, 1), (0, 0)))

    xb = _block_s1(xb, layer1_first_w1, layer1_first_w2, layer1_first_w3,
                   layer1_first_wd, 1)
    xb = _scan_blocks(xb, layer1_rest_w1, layer1_rest_w2, layer1_rest_w3, 1)
    xb = _block_s2(xb, layer2_first_w1, layer2_first_w2, layer2_first_w3,
                   layer2_first_wd)
    xb = _scan_blocks(xb, layer2_rest_w1, layer2_rest_w2, layer2_rest_w3, 1)
    xb = _block_s2(xb, layer3_first_w1, layer3_first_w2, layer3_first_w3,
                   layer3_first_wd)
    xb = _scan_blocks(xb, layer3_rest_w1, layer3_rest_w2, layer3_rest_w3, 1)
    xb = _block_s1(xb, layer4_first_w1, layer4_first_w2, layer4_first_w3,
                   layer4_first_wd, 1)
    xb = _scan_blocks(xb, layer4_rest_w1, layer4_rest_w2, layer4_rest_w3, 2)

    Nh, Hh, Wh, Ch = xb.shape
    Mh = Nh * Hh * Wh
    xfl = xb.reshape(Mh, Ch)

    y1, st1 = _mm(xfl, _w1x1(aspp_a1, Ch))
    ys = [y1]
    sts = [st1]
    for wdil, dil in ((aspp_a2, 6), (aspp_a3, 12), (aspp_a4, 18)):
        Ad = _im2col_d(xb, Ch, dil)
        yi, sti = _mm(Ad, _wk(wdil, Ch))
        ys.append(yi)
        sts.append(sti)
    xm = jnp.mean(xb.astype(jnp.float32), axis=(1, 2)).astype(jnp.bfloat16)
    y5, st5 = _mm(xm, _w1x1(aspp_a5, Ch))
    ys.append(jnp.broadcast_to(y5[:, None, :], (Nh, Hh * Wh, y5.shape[1]))
              .reshape(Mh, y5.shape[1]))
    sts.append(st5)

    depth = y1.shape[1]
    sc, hc = [], []
    for st, m in zip(sts, [Mh, Mh, Mh, Mh, Nh]):
        si, hi = _bn_affine(st, m)
        sc.append(si)
        hc.append(hi)
    Ac = jnp.concatenate(ys, axis=1)
    yc, stc = _mm(Ac, _w1x1(aspp_c2, Ac.shape[1]),
                  in_scale=jnp.concatenate(sc), in_shift=jnp.concatenate(hc))
    s6, h6 = _bn_affine(stc, Mh)
    logits = _mm(yc, _w1x1(aspp_c3, yc.shape[1]), in_scale=s6, in_shift=h6,
                 bn=False, bias=aspp_c3b)
    ncls = aspp_c3.shape[0]
    logits = logits[:, :ncls].reshape(Nh, Hh, Wh, ncls)

    Rh = _interp_matrix(H, Hh)
    Rw = _interp_matrix(W, Wh)
    X1 = jnp.transpose(logits, (1, 0, 2, 3)).reshape(Hh, N * Wh * ncls)
    T1 = _mm(Rh, X1, bn=False, op_dtype=jnp.float32)[:, :N * Wh * ncls]
    T1 = jnp.transpose(T1.reshape(H, N, Wh, ncls), (2, 1, 0, 3))
    X2 = T1.reshape(Wh, N * H * ncls)
    T2 = _mm(Rw, X2, bn=False, op_dtype=jnp.float32)[:, :N * H * ncls]
    out = jnp.transpose(T2.reshape(W, N, H, ncls), (1, 3, 2, 0))
    return out
